# initial kernel scaffold (unmeasured)
import jax
import jax.numpy as jnp
from jax import lax
from jax.experimental import pallas as pl
from jax.experimental.pallas import tpu as pltpu

N_DEV = 4
M_PER = 2048
K_BLK = 2048
N_TOT = 4096
NH = 2
KSUB = 2
KCH = K_BLK // KSUB
N_HALF = N_TOT // NH


def kernel(x, w_mat):
    def body(x_ref, w_ref, o_ref, xg, xb, load_sem, send_sems, recv_sems):
        nh = pl.program_id(0)
        j = pl.program_id(1)
        ks = pl.program_id(2)
        me = lax.axis_index("i")

        def mk_send(off):
            t = lax.rem(me + off, N_DEV)
            return pltpu.make_async_remote_copy(
                src_ref=x_ref.at[pl.ds(t * M_PER, M_PER), :],
                dst_ref=xg.at[me],
                send_sem=send_sems.at[off - 1],
                recv_sem=recv_sems.at[me],
                device_id=(t,),
                device_id_type=pl.DeviceIdType.MESH,
            )

        def mk_recv(src):
            return pltpu.make_async_remote_copy(
                src_ref=xg.at[src],
                dst_ref=xg.at[src],
                send_sem=send_sems.at[0],
                recv_sem=recv_sems.at[src],
                device_id=(src,),
                device_id_type=pl.DeviceIdType.MESH,
            )

        @pl.when((nh == 0) & (j == 0) & (ks == 0))
        def _():
            barrier_sem = pltpu.get_barrier_semaphore()
            for off in (1, 2, 3):
                t = lax.rem(me + off, N_DEV)
                pl.semaphore_signal(
                    barrier_sem, inc=1,
                    device_id=(t,), device_id_type=pl.DeviceIdType.MESH,
                )
            pl.semaphore_wait(barrier_sem, 3)
            for off in (1, 2, 3):
                mk_send(off).start()
            cp = pltpu.make_async_copy(
                x_ref.at[pl.ds(me * M_PER, M_PER), :], xg.at[me], load_sem
            )
            cp.start()
            cp.wait()

        @pl.when((nh == 0) & (ks == 0) & (j != me))
        def _():
            mk_recv(j).wait_recv()

        cp = pltpu.make_async_copy(
            xg.at[j, :, pl.ds(ks * KCH, KCH)], xb, load_sem
        )
        cp.start()
        cp.wait()

        @pl.when((j == 0) & (ks == 0))
        def _():
            o_ref[...] = jnp.zeros_like(o_ref)

        o_ref[...] += jnp.dot(
            xb[...], w_ref[...], preferred_element_type=jnp.float32
        )

        @pl.when((j == N_DEV - 1) & (ks == KSUB - 1))
        def _():
            o_ref[...] = jnp.maximum(o_ref[...], 0.0)

        @pl.when((nh == NH - 1) & (j == N_DEV - 1) & (ks == KSUB - 1))
        def _():
            for off in (1, 2, 3):
                mk_send(off).wait_send()

    return pl.pallas_call(
        body,
        grid=(NH, N_DEV, KSUB),
        out_shape=jax.ShapeDtypeStruct((M_PER, N_TOT), jnp.float32),
        in_specs=[
            pl.BlockSpec(memory_space=pl.ANY),
            pl.BlockSpec((KCH, N_HALF), lambda nh, j, ks: (j * KSUB + ks, nh)),
        ],
        out_specs=pl.BlockSpec((M_PER, N_HALF), lambda nh, j, ks: (0, nh)),
        scratch_shapes=[
            pltpu.HBM((N_DEV, M_PER, K_BLK), jnp.float32),
            pltpu.VMEM((M_PER, KCH), jnp.float32),
            pltpu.SemaphoreType.DMA,
            pltpu.SemaphoreType.DMA((3,)),
            pltpu.SemaphoreType.DMA((N_DEV,)),
        ],
        compiler_params=pltpu.CompilerParams(collective_id=0),
    )(x, w_mat)


# baseline (device time: 816490 ns/iter reference)
import jax
import jax.numpy as jnp
from jax import lax
from jax.experimental import pallas as pl
from jax.experimental.pallas import tpu as pltpu

N_DEV = 4
M_PER = 2048
K_BLK = 2048
N_TOT = 4096
NH = 2
KSUB = 4
KCH = K_BLK // KSUB
N_HALF = N_TOT // NH


def kernel(x, w_mat):
    def body(x_ref, w_ref, o_ref, xg, xb, load_sem, send_sems, recv_sems):
        nh = pl.program_id(0)
        j = pl.program_id(1)
        ks = pl.program_id(2)
        me = lax.axis_index("i")

        def mk_send(off):
            t = lax.rem(me + off, N_DEV)
            return pltpu.make_async_remote_copy(
                src_ref=x_ref.at[pl.ds(t * M_PER, M_PER), :],
                dst_ref=xg.at[me],
                send_sem=send_sems.at[off - 1],
                recv_sem=recv_sems.at[me],
                device_id=(t,),
                device_id_type=pl.DeviceIdType.MESH,
            )

        def mk_recv(src):
            return pltpu.make_async_remote_copy(
                src_ref=xg.at[src],
                dst_ref=xg.at[src],
                send_sem=send_sems.at[0],
                recv_sem=recv_sems.at[src],
                device_id=(src,),
                device_id_type=pl.DeviceIdType.MESH,
            )

        @pl.when((nh == 0) & (j == 0) & (ks == 0))
        def _():
            barrier_sem = pltpu.get_barrier_semaphore()
            for off in (1, 2, 3):
                t = lax.rem(me + off, N_DEV)
                pl.semaphore_signal(
                    barrier_sem, inc=1,
                    device_id=(t,), device_id_type=pl.DeviceIdType.MESH,
                )
            pl.semaphore_wait(barrier_sem, 3)
            for off in (1, 2, 3):
                mk_send(off).start()
            cp = pltpu.make_async_copy(
                x_ref.at[pl.ds(me * M_PER, M_PER), :], xg.at[me], load_sem
            )
            cp.start()
            cp.wait()

        @pl.when((nh == 0) & (ks == 0) & (j != me))
        def _():
            mk_recv(j).wait_recv()

        cp = pltpu.make_async_copy(
            xg.at[j, :, pl.ds(ks * KCH, KCH)], xb, load_sem
        )
        cp.start()
        cp.wait()

        @pl.when((j == 0) & (ks == 0))
        def _():
            o_ref[...] = jnp.zeros_like(o_ref)

        o_ref[...] += jnp.dot(
            xb[...], w_ref[...], preferred_element_type=jnp.float32
        )

        @pl.when((j == N_DEV - 1) & (ks == KSUB - 1))
        def _():
            o_ref[...] = jnp.maximum(o_ref[...], 0.0)

        @pl.when((nh == NH - 1) & (j == N_DEV - 1) & (ks == KSUB - 1))
        def _():
            for off in (1, 2, 3):
                mk_send(off).wait_send()

    out, _ = pl.pallas_call(
        body,
        grid=(NH, N_DEV, KSUB),
        out_shape=[
            jax.ShapeDtypeStruct((M_PER, N_TOT), jnp.float32),
            jax.ShapeDtypeStruct((N_DEV, M_PER, K_BLK), jnp.float32),
        ],
        in_specs=[
            pl.BlockSpec(memory_space=pl.ANY),
            pl.BlockSpec((KCH, N_HALF), lambda nh, j, ks: (j * KSUB + ks, nh)),
        ],
        out_specs=[
            pl.BlockSpec((M_PER, N_HALF), lambda nh, j, ks: (0, nh)),
            pl.BlockSpec(memory_space=pl.ANY),
        ],
        scratch_shapes=[
            pltpu.VMEM((M_PER, KCH), jnp.float32),
            pltpu.SemaphoreType.DMA,
            pltpu.SemaphoreType.DMA((3,)),
            pltpu.SemaphoreType.DMA((N_DEV,)),
        ],
        compiler_params=pltpu.CompilerParams(
            collective_id=0,
            vmem_limit_bytes=60 * 1024 * 1024,
        ),
    )(x, w_mat)
    return out


# device time: 541938 ns/iter; 1.5066x vs baseline; 1.5066x over previous
import jax
import jax.numpy as jnp
from jax import lax
from jax.experimental import pallas as pl
from jax.experimental.pallas import tpu as pltpu

N_DEV = 4
M_PER = 2048
K_BLK = 2048
N_TOT = 4096
KSUB = 4
KCH = K_BLK // KSUB
N_STEPS = N_DEV * KSUB


def kernel(x, w_mat):
    x16 = x.astype(jnp.bfloat16)

    def body(x_ref, w_ref, o_ref, xg, xb, wb, lsem, xsems, wsems,
             send_sems, recv_sems):
        j = pl.program_id(0)
        ks = pl.program_id(1)
        s = j * KSUB + ks
        me = lax.axis_index("i")

        def src_of(jj):
            return lax.rem(me + lax.bitwise_xor(jj, jj // 2), N_DEV)

        def mk_send(off, slot):
            t = lax.rem(me + off, N_DEV)
            return pltpu.make_async_remote_copy(
                src_ref=x_ref.at[pl.ds(t * M_PER, M_PER), :],
                dst_ref=xg.at[me],
                send_sem=send_sems.at[slot],
                recv_sem=recv_sems.at[me],
                device_id=(t,),
                device_id_type=pl.DeviceIdType.MESH,
            )

        def mk_recv(src):
            return pltpu.make_async_remote_copy(
                src_ref=xg.at[src],
                dst_ref=xg.at[src],
                send_sem=send_sems.at[0],
                recv_sem=recv_sems.at[src],
                device_id=(src,),
                device_id_type=pl.DeviceIdType.MESH,
            )

        def mk_loads(jj, kk, slot):
            src = src_of(jj)
            cpx = pltpu.make_async_copy(
                xg.at[src, :, pl.ds(kk * KCH, KCH)], xb.at[slot],
                xsems.at[slot],
            )
            cpw = pltpu.make_async_copy(
                w_ref.at[pl.ds(src * K_BLK + kk * KCH, KCH), :], wb.at[slot],
                wsems.at[slot],
            )
            return cpx, cpw

        slot = lax.rem(s, 2)

        @pl.when(s == 0)
        def _():
            barrier_sem = pltpu.get_barrier_semaphore()
            for off in (1, 2, 3):
                t = lax.rem(me + off, N_DEV)
                pl.semaphore_signal(
                    barrier_sem, inc=1,
                    device_id=(t,), device_id_type=pl.DeviceIdType.MESH,
                )
            pl.semaphore_wait(barrier_sem, 3)
            mk_send(1, 0).start()
            mk_send(3, 1).start()
            mk_send(2, 2).start()
            cp = pltpu.make_async_copy(
                x_ref.at[pl.ds(me * M_PER, M_PER), :], xg.at[me], lsem
            )
            cp.start()
            cp.wait()
            cpx, cpw = mk_loads(0, 0, 0)
            cpx.start()
            cpw.start()

        @pl.when(s < N_STEPS - 1)
        def _():
            sn = s + 1
            jn = sn // KSUB
            kn = lax.rem(sn, KSUB)

            @pl.when((kn == 0) & (jn > 0))
            def _():
                mk_recv(src_of(jn)).wait_recv()

            cpx, cpw = mk_loads(jn, kn, 1 - slot)
            cpx.start()
            cpw.start()

        cpx, cpw = mk_loads(j, ks, slot)
        cpx.wait()
        cpw.wait()

        NT = 1024
        for nt in range(N_TOT // NT):
            cols = slice(nt * NT, (nt + 1) * NT)
            val = jnp.dot(
                xb[slot],
                wb[slot, :, cols].astype(jnp.bfloat16),
                preferred_element_type=jnp.float32,
            )

            @pl.when(s == 0)
            def _():
                o_ref[:, cols] = val

            @pl.when(s > 0)
            def _():
                o_ref[:, cols] += val

        @pl.when(s == N_STEPS - 1)
        def _():
            for nt in range(N_TOT // NT):
                cols = slice(nt * NT, (nt + 1) * NT)
                o_ref[:, cols] = jnp.maximum(o_ref[:, cols], 0.0)
            for off, sl in ((1, 0), (3, 1), (2, 2)):
                mk_send(off, sl).wait_send()

    out, _ = pl.pallas_call(
        body,
        grid=(N_DEV, KSUB),
        out_shape=[
            jax.ShapeDtypeStruct((M_PER, N_TOT), jnp.float32),
            jax.ShapeDtypeStruct((N_DEV, M_PER, K_BLK), jnp.bfloat16),
        ],
        in_specs=[
            pl.BlockSpec(memory_space=pl.ANY),
            pl.BlockSpec(memory_space=pl.ANY),
        ],
        out_specs=[
            pl.BlockSpec((M_PER, N_TOT), lambda j, ks: (0, 0)),
            pl.BlockSpec(memory_space=pl.ANY),
        ],
        scratch_shapes=[
            pltpu.VMEM((2, M_PER, KCH), jnp.bfloat16),
            pltpu.VMEM((2, KCH, N_TOT), jnp.float32),
            pltpu.SemaphoreType.DMA,
            pltpu.SemaphoreType.DMA((2,)),
            pltpu.SemaphoreType.DMA((2,)),
            pltpu.SemaphoreType.DMA((3,)),
            pltpu.SemaphoreType.DMA((N_DEV,)),
        ],
        compiler_params=pltpu.CompilerParams(
            collective_id=0,
            vmem_limit_bytes=60 * 1024 * 1024,
        ),
    )(x16, w_mat)
    return out
